# Initial kernel scaffold; baseline (speedup 1.0000x reference)
import jax
import jax.numpy as jnp
from jax.experimental import pallas as pl


def kernel(text, emb_table, fc_w, fc_b):
    raise NotImplementedError("write your pallas kernel here")


# speedup vs baseline: 6.5770x; 6.5770x over previous
import jax
import jax.numpy as jnp
from jax import lax
from jax.experimental import pallas as pl
from jax.experimental.pallas import tpu as pltpu
from jax.experimental.pallas import tpu_sc as plsc

_VOCAB = 100000
_D = 300
_B = 4096
_L = 200
_C = 2
_DP = 8

_NC = 2
_NS = 16
_NW = _NC * _NS
_ROWS_W = _B // _NW
_GROUPS = _ROWS_W // 16
_KCH = _L // 8
_NCH = _GROUPS * _KCH
_CH = 128
_NSLOT = 8

_VBLK = 2000


def _proj_body(emb_ref, fcw_ref, fcb_ref, out_ref):
    p = lax.dot_general(emb_ref[...], fcw_ref[...], (((1,), (1,)), ((), ())),
                        preferred_element_type=jnp.float32)
    out_ref[...] = (p + fcb_ref[...]) * (1.0 / _L)


def _project(emb, fcw8, fcb8):
    return pl.pallas_call(
        _proj_body,
        grid=(_VOCAB // _VBLK,),
        in_specs=[
            pl.BlockSpec((_VBLK, _D), lambda i: (i, 0)),
            pl.BlockSpec((_DP, _D), lambda i: (0, 0)),
            pl.BlockSpec((1, _DP), lambda i: (0, 0)),
        ],
        out_specs=pl.BlockSpec((_VBLK, _DP), lambda i: (i, 0)),
        out_shape=jax.ShapeDtypeStruct((_VOCAB, _DP), jnp.float32),
    )(emb, fcw8, fcb8)


def _sc_body(p_hbm, idx_hbm, lane_hbm, out_hbm,
             idx_v, rows_v, lane_v, out_v, *sems):
    wid = lax.axis_index("s") * _NC + lax.axis_index("c")
    pltpu.sync_copy(idx_hbm.at[wid], idx_v)
    pltpu.sync_copy(lane_hbm, lane_v)

    pairidx = [lane_v[v] for v in range(16)]
    cidx = lane_v[16]

    def _gather(j, s):
        return pltpu.make_async_copy(
            p_hbm.at[idx_v.at[j]], rows_v.at[s], sems[s])

    for s in range(_NSLOT):
        _gather(s, s).start()

    def _outer(j0, c):
        for s in range(_NSLOT):
            j = j0 * _NSLOT + s
            _gather(j, s).wait()
            row = rows_v.at[s]
            a0 = plsc.load_gather(row, [pairidx[0], cidx])
            a1 = plsc.load_gather(row, [pairidx[1], cidx])
            for v in range(2, 16, 2):
                a0 = a0 + plsc.load_gather(row, [pairidx[v], cidx])
                a1 = a1 + plsc.load_gather(row, [pairidx[v + 1], cidx])
            g = j // _KCH
            k = j - g * _KCH
            o0 = pl.ds(g * 32, 16)
            o1 = pl.ds(g * 32 + 16, 16)

            @pl.when(k == 0)
            def _():
                out_v[o0] = a0
                out_v[o1] = a1

            @pl.when(k != 0)
            def _():
                out_v[o0] = out_v[o0] + a0
                out_v[o1] = out_v[o1] + a1

            @pl.when(j + _NSLOT < _NCH)
            def _():
                _gather(j + _NSLOT, s).start()
        return c

    lax.fori_loop(0, _NCH // _NSLOT, _outer, 0)
    pltpu.sync_copy(out_v, out_hbm.at[wid])


_lookup = pl.kernel(
    _sc_body,
    out_type=jax.ShapeDtypeStruct((_NW, _ROWS_W * _C), jnp.float32),
    mesh=plsc.VectorSubcoreMesh(core_axis_name="c", subcore_axis_name="s"),
    compiler_params=pltpu.CompilerParams(
        needs_layout_passes=False, use_tc_tiling_on_sc=False),
    scratch_types=[
        pltpu.VMEM((_NCH, _CH), jnp.int32),
        pltpu.VMEM((_NSLOT, _CH, _DP), jnp.float32),
        pltpu.VMEM((17, 16), jnp.int32),
        pltpu.VMEM((_ROWS_W * _C,), jnp.float32),
    ] + [pltpu.SemaphoreType.DMA] * _NSLOT,
)

_LANE_TAB = [[(v * 16 + i) // 2 for i in range(16)] for v in range(16)]
_LANE_TAB.append([i % 2 for i in range(16)])


def kernel(text, emb_table, fc_w, fc_b):
    fcw8 = jnp.pad(fc_w.astype(jnp.float32), ((0, _DP - _C), (0, 0)))
    fcb8 = jnp.pad(fc_b.astype(jnp.float32).reshape(1, _C),
                   ((0, 0), (0, _DP - _C)))
    p = _project(emb_table, fcw8, fcb8)
    t = text.astype(jnp.int32).reshape(_NW, _GROUPS, 16, _KCH, 8)
    idx = t.transpose(0, 1, 3, 4, 2).reshape(_NW, _NCH, _CH)
    lane_tab = jnp.asarray(_LANE_TAB, jnp.int32)
    out32 = _lookup(p, idx, lane_tab)
    return out32.reshape(_B, _C)


# speedup vs baseline: 14.0891x; 2.1422x over previous
import jax
import jax.numpy as jnp
from jax import lax
from jax.experimental import pallas as pl
from jax.experimental.pallas import tpu as pltpu
from jax.experimental.pallas import tpu_sc as plsc

_VOCAB = 100000
_D = 300
_B = 4096
_L = 200
_C = 2

_NC = 2
_NS = 16
_NW = _NC * _NS
_ROWS_W = _B // _NW
_GROUPS = _ROWS_W // 16
_KCH = _L // 8
_NCH = _GROUPS * _KCH
_CH = 128
_NSLOT = 8

_VBLK = 2048


def _proj_body(embT_ref, fcw_ref, fcb_ref, out_ref):
    p = lax.dot_general(fcw_ref[...], embT_ref[...], (((1,), (0,)), ((), ())),
                        preferred_element_type=jnp.float32)
    p = (p + fcb_ref[...]) * (1.0 / _L)
    u = lax.bitcast_convert_type(
        p.astype(jnp.bfloat16), jnp.uint16).astype(jnp.uint32)
    out_ref[...] = lax.bitcast_convert_type((u[0] << 16) | u[1], jnp.int32)


def _project(embT, fcw, fcb):
    return pl.pallas_call(
        _proj_body,
        grid=(pl.cdiv(_VOCAB, _VBLK),),
        in_specs=[
            pl.BlockSpec((_D, _VBLK), lambda i: (0, i)),
            pl.BlockSpec((_C, _D), lambda i: (0, 0)),
            pl.BlockSpec((_C, 1), lambda i: (0, 0)),
        ],
        out_specs=pl.BlockSpec((_VBLK,), lambda i: (i,)),
        out_shape=jax.ShapeDtypeStruct((_VOCAB,), jnp.int32),
    )(embT, fcw, fcb)


def _sc_body(p_hbm, idx_hbm, lane_hbm, out_hbm,
             idx_v, rows_v, lane_v, out_v, *sems):
    wid = lax.axis_index("s") * _NC + lax.axis_index("c")
    pltpu.sync_copy(idx_hbm.at[wid], idx_v)
    pltpu.sync_copy(lane_hbm, lane_v)

    lane16 = [lane_v[v] for v in range(8)]
    himask = lane_v[8]
    sh16 = lane_v[9]

    def _gather(j, s):
        return pltpu.make_async_copy(
            p_hbm.at[idx_v.at[j]], rows_v.at[s], sems[s])

    for s in range(_NSLOT):
        _gather(s, s).start()

    def _outer(j0, c):
        for s in range(_NSLOT):
            j = j0 * _NSLOT + s
            _gather(j, s).wait()
            row = rows_v.at[s]
            uv = plsc.load_gather(row, [lane16[0]])
            a0 = plsc.bitcast(uv & himask, jnp.float32)
            a1 = plsc.bitcast(lax.shift_left(uv, sh16), jnp.float32)
            for v in range(1, 8):
                uv = plsc.load_gather(row, [lane16[v]])
                a0 = a0 + plsc.bitcast(uv & himask, jnp.float32)
                a1 = a1 + plsc.bitcast(lax.shift_left(uv, sh16), jnp.float32)
            g = j // _KCH
            k = j - g * _KCH
            o0 = pl.ds(g * 16, 16)
            o1 = pl.ds(128 + g * 16, 16)

            @pl.when(k == 0)
            def _():
                out_v[o0] = a0
                out_v[o1] = a1

            @pl.when(k != 0)
            def _():
                out_v[o0] = out_v[o0] + a0
                out_v[o1] = out_v[o1] + a1

            @pl.when(j + _NSLOT < _NCH)
            def _():
                _gather(j + _NSLOT, s).start()
        return c

    lax.fori_loop(0, _NCH // _NSLOT, _outer, 0)
    pltpu.sync_copy(out_v, out_hbm.at[wid])


_lookup = pl.kernel(
    _sc_body,
    out_type=jax.ShapeDtypeStruct((_NW, _C * _ROWS_W), jnp.float32),
    mesh=plsc.VectorSubcoreMesh(core_axis_name="c", subcore_axis_name="s"),
    compiler_params=pltpu.CompilerParams(
        needs_layout_passes=False, use_tc_tiling_on_sc=False),
    scratch_types=[
        pltpu.VMEM((_NCH, _CH), jnp.int32),
        pltpu.VMEM((_NSLOT, _CH), jnp.int32),
        pltpu.VMEM((10, 16), jnp.int32),
        pltpu.VMEM((_C * _ROWS_W,), jnp.float32),
    ] + [pltpu.SemaphoreType.DMA] * _NSLOT,
)

_LANE_TAB = [[v * 16 + i for i in range(16)] for v in range(8)]
_LANE_TAB.append([-65536] * 16)
_LANE_TAB.append([16] * 16)


def kernel(text, emb_table, fc_w, fc_b):
    embT = emb_table.T
    ptab = _project(embT, fc_w.astype(jnp.float32),
                    fc_b.astype(jnp.float32).reshape(_C, 1))
    textT = text.astype(jnp.int32).T
    u = textT.reshape(_KCH, 8, _NW, _GROUPS, 16)
    idx = u.transpose(2, 3, 0, 1, 4).reshape(_NW, _NCH, _CH)
    lane_tab = jnp.asarray(_LANE_TAB, jnp.int32)
    out32 = _lookup(ptab, idx, lane_tab)
    return out32.reshape(_NW, _C, _ROWS_W).transpose(0, 2, 1).reshape(_B, _C)


# speedup vs baseline: 15.6527x; 1.1110x over previous
import jax
import jax.numpy as jnp
from jax import lax
from jax.experimental import pallas as pl
from jax.experimental.pallas import tpu as pltpu
from jax.experimental.pallas import tpu_sc as plsc

_VOCAB = 100000
_D = 300
_B = 4096
_L = 200
_C = 2

_NC = 2
_NS = 16
_NW = _NC * _NS
_ROWS_W = _B // _NW
_GROUPS = _ROWS_W // 16
_KCH = _L // 8
_NCH = _GROUPS * _KCH
_CH = 128
_NSLOT = 8

_VBLK = 8192


def _proj_body(embT_ref, fcw_ref, fcb_ref, out_ref):
    p = lax.dot_general(fcw_ref[...], embT_ref[...], (((1,), (0,)), ((), ())),
                        preferred_element_type=jnp.float32)
    p = (p + fcb_ref[...]) * (1.0 / _L)
    u = lax.bitcast_convert_type(
        p.astype(jnp.bfloat16), jnp.uint16).astype(jnp.uint32)
    out_ref[...] = lax.bitcast_convert_type((u[0] << 16) | u[1], jnp.int32)


def _project(embT, fcw, fcb):
    return pl.pallas_call(
        _proj_body,
        grid=(pl.cdiv(_VOCAB, _VBLK),),
        in_specs=[
            pl.BlockSpec((_D, _VBLK), lambda i: (0, i)),
            pl.BlockSpec((_C, _D), lambda i: (0, 0)),
            pl.BlockSpec((_C, 1), lambda i: (0, 0)),
        ],
        out_specs=pl.BlockSpec((_VBLK,), lambda i: (i,)),
        out_shape=jax.ShapeDtypeStruct((_VOCAB,), jnp.int32),
    )(embT, fcw, fcb)


def _sc_body(p_hbm, idx_hbm, lane_hbm, out_hbm,
             idx_v, rows_v, lane_v, out_v, *sems):
    wid = lax.axis_index("s") * _NC + lax.axis_index("c")
    pltpu.sync_copy(idx_hbm.at[wid], idx_v)
    pltpu.sync_copy(lane_hbm, lane_v)

    lane16 = [lane_v[v] for v in range(8)]
    himask = lane_v[8]
    sh16 = lane_v[9]

    def _gather(j, s):
        return pltpu.make_async_copy(
            p_hbm.at[idx_v.at[j]], rows_v.at[s], sems[s])

    for s in range(_NSLOT):
        _gather(s, s).start()

    def _outer(j0, c):
        for s in range(_NSLOT):
            j = j0 * _NSLOT + s
            _gather(j, s).wait()
            row = rows_v.at[s]
            uv = plsc.load_gather(row, [lane16[0]])
            a0 = plsc.bitcast(uv & himask, jnp.float32)
            a1 = plsc.bitcast(lax.shift_left(uv, sh16), jnp.float32)
            for v in range(1, 8):
                uv = plsc.load_gather(row, [lane16[v]])
                a0 = a0 + plsc.bitcast(uv & himask, jnp.float32)
                a1 = a1 + plsc.bitcast(lax.shift_left(uv, sh16), jnp.float32)
            g = j // _KCH
            k = j - g * _KCH
            o0 = pl.ds(g * 16, 16)
            o1 = pl.ds(128 + g * 16, 16)

            @pl.when(k == 0)
            def _():
                out_v[o0] = a0
                out_v[o1] = a1

            @pl.when(k != 0)
            def _():
                out_v[o0] = out_v[o0] + a0
                out_v[o1] = out_v[o1] + a1

            @pl.when(j + _NSLOT < _NCH)
            def _():
                _gather(j + _NSLOT, s).start()
        return c

    lax.fori_loop(0, _NCH // _NSLOT, _outer, 0)
    pltpu.sync_copy(out_v, out_hbm.at[wid])


_lookup = pl.kernel(
    _sc_body,
    out_type=jax.ShapeDtypeStruct((_NW, _C * _ROWS_W), jnp.float32),
    mesh=plsc.VectorSubcoreMesh(core_axis_name="c", subcore_axis_name="s"),
    compiler_params=pltpu.CompilerParams(
        needs_layout_passes=False, use_tc_tiling_on_sc=False),
    scratch_types=[
        pltpu.VMEM((_NCH, _CH), jnp.int32),
        pltpu.VMEM((_NSLOT, _CH), jnp.int32),
        pltpu.VMEM((10, 16), jnp.int32),
        pltpu.VMEM((_C * _ROWS_W,), jnp.float32),
    ] + [pltpu.SemaphoreType.DMA] * _NSLOT,
)

_LANE_TAB = [[v * 16 + i for i in range(16)] for v in range(8)]
_LANE_TAB.append([-65536] * 16)
_LANE_TAB.append([16] * 16)


def kernel(text, emb_table, fc_w, fc_b):
    embT = emb_table.T
    ptab = _project(embT, fc_w.astype(jnp.float32),
                    fc_b.astype(jnp.float32).reshape(_C, 1))
    textT = text.astype(jnp.int32).T
    u = textT.reshape(_KCH, 8, _NW, _GROUPS, 16)
    idx = u.transpose(2, 3, 0, 1, 4).reshape(_NW, _NCH, _CH)
    lane_tab = jnp.asarray(_LANE_TAB, jnp.int32)
    out32 = _lookup(ptab, idx, lane_tab)
    return out32.reshape(_NW, _C, _ROWS_W).transpose(0, 2, 1).reshape(_B, _C)
